# baseline (speedup 1.0000x reference)
import jax
import jax.numpy as jnp
from jax import lax
from jax.experimental import pallas as pl
from jax.experimental.pallas import tpu as pltpu
from jax.experimental.pallas import tpu_sc as plsc

N_ROWS = 1000000
N_FACTORS = 64
BATCH = 16384
NC = 2
NS = 16
NW = NC * NS
B_PER_W = BATCH // NW
IDX_CHUNK = 128
N_CHUNKS = B_PER_W // IDX_CHUNK
L = 16
FCH = N_FACTORS // L
GPC = IDX_CHUNK // L
PACKED_W = 2 * N_FACTORS


def _gmf_body(uids_ref, iids_ref, utab_ref, itab_ref, w_ref, b_ref,
              out_ref, uid_v, iid_v, uq_v, iq_v, upar_v, ipar_v,
              u_rows, i_rows, w_v, b_v, out_v, tr_v, sems):
    wid = lax.axis_index("s") * NC + lax.axis_index("c")

    pltpu.sync_copy(uids_ref.at[pl.ds(wid * N_CHUNKS, N_CHUNKS)], uid_v)
    pltpu.sync_copy(iids_ref.at[pl.ds(wid * N_CHUNKS, N_CHUNKS)], iid_v)
    pltpu.sync_copy(w_ref, w_v)
    pltpu.sync_copy(b_ref, b_v)

    lane = lax.iota(jnp.int32, L)

    for k in range(N_CHUNKS):
        for m in range(IDX_CHUNK // L):
            u = uid_v[k, pl.ds(m * L, L)]
            i = iid_v[k, pl.ds(m * L, L)]
            uq_v[k, pl.ds(m * L, L)] = u >> 1
            iq_v[k, pl.ds(m * L, L)] = i >> 1
            upar_v[pl.ds(k * IDX_CHUNK + m * L, L)] = (u & 1) << 6
            ipar_v[pl.ds(k * IDX_CHUNK + m * L, L)] = (i & 1) << 6

    def fire(k):
        s = sems.at[k % 2]
        return (pltpu.async_copy(utab_ref.at[uq_v.at[k]],
                                 u_rows.at[k % 2], s),
                pltpu.async_copy(itab_ref.at[iq_v.at[k]],
                                 i_rows.at[k % 2], s))

    w_regs = [w_v[pl.ds(c * L, L)] for c in range(FCH)]
    b_vec = b_v[...]

    inflight = fire(0)
    for k in range(N_CHUNKS):
        for c in inflight:
            c.wait()
        if k + 1 < N_CHUNKS:
            inflight = fire(k + 1)
        bu = u_rows.at[k % 2]
        bi = i_rows.at[k % 2]

        def group_body(g, carry, k=k, bu=bu, bi=bi):
            r0 = g * L
            for j in range(L):
                r = r0 + j
                gr = k * IDX_CHUNK + r
                rfull = jnp.full((L,), r, jnp.int32)
                cu = plsc.load_gather(upar_v, [jnp.full((L,), gr, jnp.int32)]) + lane
                ci = plsc.load_gather(ipar_v, [jnp.full((L,), gr, jnp.int32)]) + lane
                acc = None
                for c in range(FCH):
                    uv = plsc.load_gather(bu, [rfull, cu + c * L])
                    iv = plsc.load_gather(bi, [rfull, ci + c * L])
                    t = uv * iv * w_regs[c]
                    acc = t if acc is None else acc + t
                plsc.store_scatter(tr_v, [lane * L + j], acc)
            s = b_vec
            for l in range(L):
                s = s + tr_v[pl.ds(l * L, L)]
            out_v[pl.ds(k * IDX_CHUNK + r0, L)] = s
            return carry

        lax.fori_loop(0, GPC, group_body, 0)

    pltpu.sync_copy(out_v, out_ref.at[pl.ds(wid * B_PER_W, B_PER_W)])


@jax.jit
def _gmf(uids, iids, user_table, item_table, fc_w, fc_b):
    uids2 = uids.reshape(NW * N_CHUNKS, IDX_CHUNK)
    iids2 = iids.reshape(NW * N_CHUNKS, IDX_CHUNK)
    utab2 = user_table.reshape(N_ROWS // 2, PACKED_W)
    itab2 = item_table.reshape(N_ROWS // 2, PACKED_W)
    w_flat = fc_w.reshape(N_FACTORS)
    b_vec = jnp.broadcast_to(fc_b, (L,))
    mesh = plsc.VectorSubcoreMesh(
        core_axis_name="c", subcore_axis_name="s",
        num_cores=NC, num_subcores=NS)
    run = pl.kernel(
        _gmf_body,
        out_type=jax.ShapeDtypeStruct((BATCH,), jnp.float32),
        mesh=mesh,
        scratch_types=[
            pltpu.VMEM((N_CHUNKS, IDX_CHUNK), jnp.int32),
            pltpu.VMEM((N_CHUNKS, IDX_CHUNK), jnp.int32),
            pltpu.VMEM((N_CHUNKS, IDX_CHUNK), jnp.int32),
            pltpu.VMEM((N_CHUNKS, IDX_CHUNK), jnp.int32),
            pltpu.VMEM((B_PER_W,), jnp.int32),
            pltpu.VMEM((B_PER_W,), jnp.int32),
            pltpu.VMEM((2, IDX_CHUNK, PACKED_W), jnp.float32),
            pltpu.VMEM((2, IDX_CHUNK, PACKED_W), jnp.float32),
            pltpu.VMEM((N_FACTORS,), jnp.float32),
            pltpu.VMEM((L,), jnp.float32),
            pltpu.VMEM((B_PER_W,), jnp.float32),
            pltpu.VMEM((L * L,), jnp.float32),
            pltpu.SemaphoreType.DMA((2,)),
        ],
        compiler_params=pltpu.CompilerParams(needs_layout_passes=False),
    )
    return run(uids2, iids2, utab2, itab2, w_flat, b_vec)


def kernel(uids, iids, user_table, item_table, fc_w, fc_b):
    return _gmf(uids, iids, user_table, item_table, fc_w, fc_b).reshape(
        BATCH, 1)
